# baseline (speedup 1.0000x reference)
import functools

import jax
import jax.numpy as jnp
from jax import lax
from jax.experimental import pallas as pl
from jax.experimental.pallas import tpu as pltpu

_SMOOTH = 0.1
_BLOCK_ROWS = 2048


def _tc_body(x_ref, t_ref, out_ref, *, classes):
    i = pl.program_id(0)
    x = x_ref[...]
    t = t_ref[0, 0, :]

    a = _SMOOTH / (classes - 1)
    b = (1.0 - _SMOOTH) - a

    m = jnp.max(x, axis=1, keepdims=True)
    e = jnp.exp(x - m)
    col = lax.broadcasted_iota(jnp.int32, x.shape, 1)
    hit = jnp.where(col == t[:, None], x, 0.0)

    ones = jnp.ones((classes, 128), jnp.float32)
    se = jax.lax.dot_general(
        e, ones, (((1,), (0,)), ((), ())),
        preferred_element_type=jnp.float32)[:, :1]
    hx = jax.lax.dot_general(
        a * x + b * hit, ones, (((1,), (0,)), ((), ())),
        preferred_element_type=jnp.float32)[:, :1]

    sum_lse = jnp.sum(m) + jnp.sum(jnp.log(se))
    part = (a * classes + b) * sum_lse - jnp.sum(hx)

    @pl.when(i == 0)
    def _init():
        out_ref[0, 0] = 0.0

    out_ref[0, 0] += part


def kernel(prediction, target):
    n, classes = prediction.shape
    grid = n // _BLOCK_ROWS
    tgt = target.astype(jnp.int32).reshape(grid, 1, _BLOCK_ROWS)

    total = pl.pallas_call(
        functools.partial(_tc_body, classes=classes),
        grid=(grid,),
        in_specs=[
            pl.BlockSpec((_BLOCK_ROWS, classes), lambda i: (i, 0)),
            pl.BlockSpec((1, 1, _BLOCK_ROWS), lambda i: (i, 0, 0)),
        ],
        out_specs=pl.BlockSpec(
            (1, 1), lambda i: (0, 0), memory_space=pltpu.SMEM
        ),
        out_shape=jax.ShapeDtypeStruct((1, 1), jnp.float32),
    )(prediction, tgt)

    return total[0, 0] / n


# speedup vs baseline: 1.1220x; 1.1220x over previous
import functools

import jax
import jax.numpy as jnp
from jax import lax
from jax.experimental import pallas as pl
from jax.experimental.pallas import tpu as pltpu

_SMOOTH = 0.1


def _tc_body(x_ref, t_ref, out_ref, *, block_rows, classes):
    i = pl.program_id(0)
    x = x_ref[...]
    m = jnp.max(x, axis=1, keepdims=True)
    se = jnp.sum(jnp.exp(x - m), axis=1)
    sum_lse = jnp.sum(m) + jnp.sum(jnp.log(se))

    t = t_ref[i, :]
    col = lax.broadcasted_iota(jnp.int32, (block_rows, classes), 1)
    a = _SMOOTH / (classes - 1)
    b = (1.0 - _SMOOTH) - a
    wx = a * jnp.sum(x) + b * jnp.sum(jnp.where(col == t[:, None], x, 0.0))

    part = (a * classes + b) * sum_lse - wx

    @pl.when(i == 0)
    def _init():
        out_ref[0, 0] = 0.0

    out_ref[0, 0] += part


def kernel(prediction, target):
    n, classes = prediction.shape
    block_rows = 2048
    grid = n // block_rows
    tgt = target.astype(jnp.int32).reshape(grid, block_rows)

    total = pl.pallas_call(
        functools.partial(_tc_body, block_rows=block_rows, classes=classes),
        grid=(grid,),
        in_specs=[
            pl.BlockSpec((block_rows, classes), lambda i: (i, 0)),
            pl.BlockSpec((grid, block_rows), lambda i: (0, 0)),
        ],
        out_specs=pl.BlockSpec(
            (1, 1), lambda i: (0, 0), memory_space=pltpu.SMEM
        ),
        out_shape=jax.ShapeDtypeStruct((1, 1), jnp.float32),
    )(prediction, tgt)

    return total[0, 0] / n


# speedup vs baseline: 1.1359x; 1.0124x over previous
import functools

import jax
import jax.numpy as jnp
from jax import lax
from jax.experimental import pallas as pl
from jax.experimental.pallas import tpu as pltpu

_SMOOTH = 0.1


def _tc_body(x_ref, t_ref, out_ref, *, block_rows, classes):
    i = pl.program_id(0)
    x = x_ref[...]
    m = jnp.max(x, axis=1, keepdims=True)
    se = jnp.sum(jnp.exp(x - m), axis=1)
    sum_lse = jnp.sum(m) + jnp.sum(jnp.log(se))

    t = t_ref[0, 0, :]
    col = lax.broadcasted_iota(jnp.int32, (block_rows, classes), 1)
    a = _SMOOTH / (classes - 1)
    b = (1.0 - _SMOOTH) - a
    wx = a * jnp.sum(x) + b * jnp.sum(jnp.where(col == t[:, None], x, 0.0))

    part = (a * classes + b) * sum_lse - wx

    @pl.when(i == 0)
    def _init():
        out_ref[0, 0] = 0.0

    out_ref[0, 0] += part


def kernel(prediction, target):
    n, classes = prediction.shape
    block_rows = 2048
    grid = n // block_rows
    tgt = target.astype(jnp.int32).reshape(grid, 1, block_rows)

    total = pl.pallas_call(
        functools.partial(_tc_body, block_rows=block_rows, classes=classes),
        grid=(grid,),
        in_specs=[
            pl.BlockSpec((block_rows, classes), lambda i: (i, 0)),
            pl.BlockSpec((1, 1, block_rows), lambda i: (i, 0, 0)),
        ],
        out_specs=pl.BlockSpec(
            (1, 1), lambda i: (0, 0), memory_space=pltpu.SMEM
        ),
        out_shape=jax.ShapeDtypeStruct((1, 1), jnp.float32),
    )(prediction, tgt)

    return total[0, 0] / n
